# baseline (speedup 1.0000x reference)
import jax
import jax.numpy as jnp
from jax import lax
from jax.experimental import pallas as pl
from jax.experimental.pallas import tpu as pltpu
from jax.experimental.pallas import tpu_sc as plsc

NC, NS, L = 2, 16, 16
NW = NC * NS

N = 10000
NPAD = 10240
D = 128
NQ = 4
DQ = D // NQ
E = 320000
CB = 128
NCH = 160
EPT = NCH * CB
EPAD = EPT * NS
JUNK = N + 16
RPT = NPAD // NS
TRT = N // NS
TGB = 125
NBUF = 8

_mesh = plsc.VectorSubcoreMesh(
    core_axis_name="c", subcore_axis_name="s", num_cores=NC, num_subcores=NS)
_sc_params = pltpu.CompilerParams(
    needs_layout_passes=False, use_tc_tiling_on_sc=False)


def _deg_body(dst_hbm, out_hbm, dst_v, deg_v):
    cid = lax.axis_index("c")
    sid = lax.axis_index("s")
    wid = cid * NS + sid
    pltpu.sync_copy(dst_hbm.at[wid], dst_v)
    zeros = jnp.zeros((L,), jnp.float32)

    @pl.loop(0, NPAD // L)
    def _z(i):
        deg_v[pl.ds(i * L, L)] = zeros

    ones = jnp.ones((L,), jnp.float32)

    @pl.loop(0, (E // NW) // L)
    def _s(i):
        idx = dst_v[pl.ds(i * L, L)]
        plsc.addupdate_scatter(deg_v, [idx], ones)

    pltpu.sync_copy(deg_v, out_hbm.at[wid])


def _deg_call(dst):
    k = pl.kernel(
        _deg_body,
        out_type=jax.ShapeDtypeStruct((NW, NPAD), jnp.float32),
        mesh=_mesh,
        compiler_params=_sc_params,
        scratch_types=[
            pltpu.VMEM((E // NW,), jnp.int32),
            pltpu.VMEM((NPAD,), jnp.float32),
        ],
    )
    return k(dst.reshape(NW, E // NW))


def _agg_body(v_hbm, src_hbm, dst_hbm, z_hbm, iot_hbm, out_hbm,
              src_v, dst_v, iot_v, rows_v, table_sh, acc_sh, gsems, ssems):
    cid = lax.axis_index("c")
    sid = lax.axis_index("s")
    pltpu.sync_copy(src_hbm.at[sid], src_v)
    pltpu.sync_copy(dst_hbm.at[sid], dst_v)
    pltpu.sync_copy(iot_hbm.at[sid], iot_v)

    for p in range(2):
        q = cid * 2 + p
        for k in range(TRT // TGB):
            pltpu.async_copy(
                v_hbm.at[q].at[iot_v.at[k]],
                rows_v.at[0, pl.ds(0, TGB)],
                gsems.at[0]).wait()
            pltpu.sync_copy(rows_v.at[0, pl.ds(0, TGB)],
                            table_sh.at[pl.ds(sid * TRT + k * TGB, TGB)])
        pltpu.sync_copy(z_hbm.at[pl.ds(sid * RPT, RPT)],
                        acc_sh.at[pl.ds(sid * RPT, RPT)])
        plsc.subcore_barrier()

        for b in range(NBUF):
            gtab = table_sh if b % 2 == 0 else v_hbm.at[q]
            pltpu.async_copy(gtab.at[src_v.at[b]], rows_v.at[b],
                             gsems.at[b])

        @pl.loop(0, NCH, step=NBUF)
        def _batches(j0):
            for b in range(NBUF):
                j = j0 + b
                gtab = table_sh if b % 2 == 0 else v_hbm.at[q]
                pltpu.make_async_copy(
                    gtab.at[src_v.at[j]], rows_v.at[b],
                    gsems.at[b]).wait()
                pltpu.async_copy(rows_v.at[b], acc_sh.at[dst_v.at[j]],
                                 ssems.at[b], add=True)

                @pl.when(j + NBUF < NCH)
                def _refill():
                    pltpu.make_async_copy(
                        rows_v.at[b], acc_sh.at[dst_v.at[j]],
                        ssems.at[b]).wait()
                    pltpu.async_copy(gtab.at[src_v.at[j + NBUF]],
                                     rows_v.at[b], gsems.at[b])

        for b in range(NBUF):
            j = NCH - NBUF + b
            pltpu.make_async_copy(
                rows_v.at[b], acc_sh.at[dst_v.at[j]], ssems.at[b]).wait()

        plsc.subcore_barrier()
        pltpu.sync_copy(acc_sh.at[pl.ds(sid * RPT, RPT)],
                        out_hbm.at[q, pl.ds(sid * RPT, RPT)])


def _agg_call(v4, srcp, dstp, zrows, iot):
    k = pl.kernel(
        _agg_body,
        out_type=jax.ShapeDtypeStruct((NQ, NPAD, DQ), jnp.float32),
        mesh=_mesh,
        compiler_params=_sc_params,
        scratch_types=[
            pltpu.VMEM((NCH, CB), jnp.int32),
            pltpu.VMEM((NCH, CB), jnp.int32),
            pltpu.VMEM((TRT // TGB, TGB), jnp.int32),
            pltpu.VMEM((NBUF, CB, DQ), jnp.float32),
            pltpu.VMEM_SHARED((N, DQ), jnp.float32),
            pltpu.VMEM_SHARED((NPAD, DQ), jnp.float32),
            pltpu.SemaphoreType.DMA((NBUF,)),
            pltpu.SemaphoreType.DMA((NBUF,)),
        ],
    )
    return k(v4, srcp, dstp, zrows, iot)


def _dinv_body(p_ref, o_ref):
    s = jnp.sum(p_ref[...], axis=0, keepdims=True)
    o_ref[...] = lax.rsqrt(1.0 + s)


def _dinv_call(parts):
    return pl.pallas_call(
        _dinv_body,
        out_shape=jax.ShapeDtypeStruct((1, NPAD), jnp.float32),
    )(parts)


def _scale_body(x_ref, d_ref, o_ref):
    xv = x_ref[...] * d_ref[...]
    for qq in range(NQ):
        o_ref[qq] = xv[:, DQ * qq:DQ * (qq + 1)]


def _scale_call(x, dcol):
    rb = 1000
    return pl.pallas_call(
        _scale_body,
        grid=(N // rb,),
        in_specs=[
            pl.BlockSpec((rb, D), lambda i: (i, 0)),
            pl.BlockSpec((rb, 1), lambda i: (i, 0)),
        ],
        out_specs=pl.BlockSpec((NQ, rb, DQ), lambda i: (0, i, 0)),
        out_shape=jax.ShapeDtypeStruct((NQ, N, DQ), jnp.float32),
    )(x, dcol)


def _mid_body(xs_ref, p_ref, d_ref, w1_ref, b1_ref, w2_ref, o_ref):
    d = d_ref[...]
    xv = jnp.concatenate([xs_ref[qq] for qq in range(NQ)], axis=1)
    pv = jnp.concatenate([p_ref[qq] for qq in range(NQ)], axis=1)
    t = d * (xv + pv)
    h = jnp.dot(t, w1_ref[...], preferred_element_type=jnp.float32)
    h = jax.nn.gelu(h + b1_ref[...])
    y = jnp.dot(h, w2_ref[...], preferred_element_type=jnp.float32)
    yd = y * d
    for qq in range(NQ):
        o_ref[qq] = yd[:, DQ * qq:DQ * (qq + 1)]


def _mid_call(xs4, p4, dcol, W1, b1, W2):
    rb = 1000
    dh = W1.shape[1]
    return pl.pallas_call(
        _mid_body,
        grid=(N // rb,),
        in_specs=[
            pl.BlockSpec((NQ, rb, DQ), lambda i: (0, i, 0)),
            pl.BlockSpec((NQ, rb, DQ), lambda i: (0, i, 0)),
            pl.BlockSpec((rb, 1), lambda i: (i, 0)),
            pl.BlockSpec((D, dh), lambda i: (0, 0)),
            pl.BlockSpec((1, dh), lambda i: (0, 0)),
            pl.BlockSpec((dh, D), lambda i: (0, 0)),
        ],
        out_specs=pl.BlockSpec((NQ, rb, DQ), lambda i: (0, i, 0)),
        out_shape=jax.ShapeDtypeStruct((NQ, N, DQ), jnp.float32),
    )(xs4, p4, dcol, W1, b1.reshape(1, dh), W2)


def _fin_body(ys_ref, q_ref, d_ref, b2_ref, o_ref):
    yv = jnp.concatenate([ys_ref[qq] for qq in range(NQ)], axis=1)
    qv = jnp.concatenate([q_ref[qq] for qq in range(NQ)], axis=1)
    t = d_ref[...] * (yv + qv)
    o_ref[...] = jax.nn.gelu(t + b2_ref[...])


def _fin_call(ys4, q4, dcol, b2):
    rb = 1000
    return pl.pallas_call(
        _fin_body,
        grid=(N // rb,),
        in_specs=[
            pl.BlockSpec((NQ, rb, DQ), lambda i: (0, i, 0)),
            pl.BlockSpec((NQ, rb, DQ), lambda i: (0, i, 0)),
            pl.BlockSpec((rb, 1), lambda i: (i, 0)),
            pl.BlockSpec((1, D), lambda i: (0, 0)),
        ],
        out_specs=pl.BlockSpec((rb, D), lambda i: (i, 0)),
        out_shape=jax.ShapeDtypeStruct((N, D), jnp.float32),
    )(ys4, q4, dcol, b2.reshape(1, D))


def kernel(x, edge_index, batch, W1, b1, W2, b2):
    src = edge_index[0]
    dst = edge_index[1]
    pad = EPAD - E
    srcp = jnp.concatenate(
        [src, jnp.zeros((pad,), jnp.int32)]).reshape(NS, NCH, CB)
    dstp = jnp.concatenate(
        [dst, jnp.full((pad,), JUNK, jnp.int32)]).reshape(NS, NCH, CB)
    zrows = jnp.zeros((NPAD, DQ), jnp.float32)
    iot = jnp.arange(N, dtype=jnp.int32).reshape(NS, TRT // TGB, TGB)
    srcp, dstp, zrows, iot = lax.optimization_barrier((srcp, dstp, zrows, iot))

    deg_parts = _deg_call(dst)
    dinv = _dinv_call(deg_parts)
    dcol = dinv.reshape(NPAD, 1)[:N]

    xs4 = _scale_call(x, dcol)
    p4 = _agg_call(xs4, srcp, dstp, zrows, iot)
    ys4 = _mid_call(xs4, p4[:, :N], dcol, W1, b1, W2)
    q4 = _agg_call(ys4, srcp, dstp, zrows, iot)
    out = _fin_call(ys4, q4[:, :N], dcol, b2)
    return (out, None)


# speedup vs baseline: 1.2506x; 1.2506x over previous
import jax
import jax.numpy as jnp
from jax import lax
from jax.experimental import pallas as pl
from jax.experimental.pallas import tpu as pltpu
from jax.experimental.pallas import tpu_sc as plsc

NC, NS, L = 2, 16, 16
NW = NC * NS

N = 10000
NPAD = 10240
D = 128
NQ = 4
DQ = D // NQ
E = 320000
CB = 128
NCH = 160
EPT = NCH * CB
EPAD = EPT * NS
JUNK = N + 16
RPT = NPAD // NS
TRT = N // NS
TGB = 125
NBUF = 8

_mesh = plsc.VectorSubcoreMesh(
    core_axis_name="c", subcore_axis_name="s", num_cores=NC, num_subcores=NS)
_sc_params = pltpu.CompilerParams(
    needs_layout_passes=False, use_tc_tiling_on_sc=False)


def _deg_body(dst_hbm, out_hbm, dst_v, deg_v):
    cid = lax.axis_index("c")
    sid = lax.axis_index("s")
    wid = cid * NS + sid
    pltpu.sync_copy(dst_hbm.at[wid], dst_v)
    zeros = jnp.zeros((L,), jnp.float32)

    @pl.loop(0, NPAD // L)
    def _z(i):
        deg_v[pl.ds(i * L, L)] = zeros

    ones = jnp.ones((L,), jnp.float32)

    @pl.loop(0, (E // NW) // L)
    def _s(i):
        idx = dst_v[pl.ds(i * L, L)]
        plsc.addupdate_scatter(deg_v, [idx], ones)

    pltpu.sync_copy(deg_v, out_hbm.at[wid])


def _deg_call(dst):
    k = pl.kernel(
        _deg_body,
        out_type=jax.ShapeDtypeStruct((NW, NPAD), jnp.float32),
        mesh=_mesh,
        compiler_params=_sc_params,
        scratch_types=[
            pltpu.VMEM((E // NW,), jnp.int32),
            pltpu.VMEM((NPAD,), jnp.float32),
        ],
    )
    return k(dst.reshape(NW, E // NW))


def _agg_body(v_hbm, src_hbm, dst_hbm, z_hbm, iot_hbm, out_hbm,
              src_v, dst_v, iot_v, rows_v, table_sh, acc_sh, gsems, ssems):
    cid = lax.axis_index("c")
    sid = lax.axis_index("s")
    pltpu.sync_copy(src_hbm.at[sid], src_v)
    pltpu.sync_copy(dst_hbm.at[sid], dst_v)
    pltpu.sync_copy(iot_hbm.at[sid], iot_v)

    for p in range(2):
        q = cid * 2 + p
        for k in range(TRT // TGB):
            pltpu.async_copy(
                v_hbm.at[q].at[iot_v.at[k]],
                rows_v.at[0, pl.ds(0, TGB)],
                gsems.at[0]).wait()
            pltpu.sync_copy(rows_v.at[0, pl.ds(0, TGB)],
                            table_sh.at[pl.ds(sid * TRT + k * TGB, TGB)])
        pltpu.sync_copy(z_hbm.at[pl.ds(sid * RPT, RPT)],
                        acc_sh.at[pl.ds(sid * RPT, RPT)])
        plsc.subcore_barrier()

        for b in range(NBUF):
            pltpu.async_copy(table_sh.at[src_v.at[b]], rows_v.at[b],
                             gsems.at[b])

        @pl.loop(0, NCH, step=NBUF)
        def _batches(j0):
            for b in range(NBUF):
                j = j0 + b
                pltpu.make_async_copy(
                    table_sh.at[src_v.at[j]], rows_v.at[b],
                    gsems.at[b]).wait()
                pltpu.async_copy(rows_v.at[b], acc_sh.at[dst_v.at[j]],
                                 ssems.at[b], add=True)

                @pl.when(j + NBUF < NCH)
                def _refill():
                    pltpu.make_async_copy(
                        rows_v.at[b], acc_sh.at[dst_v.at[j]],
                        ssems.at[b]).wait()
                    pltpu.async_copy(table_sh.at[src_v.at[j + NBUF]],
                                     rows_v.at[b], gsems.at[b])

        for b in range(NBUF):
            j = NCH - NBUF + b
            pltpu.make_async_copy(
                rows_v.at[b], acc_sh.at[dst_v.at[j]], ssems.at[b]).wait()

        plsc.subcore_barrier()
        pltpu.sync_copy(acc_sh.at[pl.ds(sid * RPT, RPT)],
                        out_hbm.at[q, pl.ds(sid * RPT, RPT)])


def _agg_call(v4, srcp, dstp, zrows, iot):
    k = pl.kernel(
        _agg_body,
        out_type=jax.ShapeDtypeStruct((NQ, NPAD, DQ), jnp.float32),
        mesh=_mesh,
        compiler_params=_sc_params,
        scratch_types=[
            pltpu.VMEM((NCH, CB), jnp.int32),
            pltpu.VMEM((NCH, CB), jnp.int32),
            pltpu.VMEM((TRT // TGB, TGB), jnp.int32),
            pltpu.VMEM((NBUF, CB, DQ), jnp.float32),
            pltpu.VMEM_SHARED((N, DQ), jnp.float32),
            pltpu.VMEM_SHARED((NPAD, DQ), jnp.float32),
            pltpu.SemaphoreType.DMA((NBUF,)),
            pltpu.SemaphoreType.DMA((NBUF,)),
        ],
    )
    return k(v4, srcp, dstp, zrows, iot)


def _dinv_body(p_ref, o_ref):
    s = jnp.sum(p_ref[...], axis=0, keepdims=True)
    o_ref[...] = lax.rsqrt(1.0 + s)


def _dinv_call(parts):
    return pl.pallas_call(
        _dinv_body,
        out_shape=jax.ShapeDtypeStruct((1, NPAD), jnp.float32),
    )(parts)


def _scale_body(x_ref, d_ref, o_ref):
    xv = x_ref[...] * d_ref[...]
    for qq in range(NQ):
        o_ref[qq] = xv[:, DQ * qq:DQ * (qq + 1)]


def _scale_call(x, dcol):
    rb = 1000
    return pl.pallas_call(
        _scale_body,
        grid=(N // rb,),
        in_specs=[
            pl.BlockSpec((rb, D), lambda i: (i, 0)),
            pl.BlockSpec((rb, 1), lambda i: (i, 0)),
        ],
        out_specs=pl.BlockSpec((NQ, rb, DQ), lambda i: (0, i, 0)),
        out_shape=jax.ShapeDtypeStruct((NQ, N, DQ), jnp.float32),
    )(x, dcol)


def _mid_body(xs_ref, p_ref, d_ref, w1_ref, b1_ref, w2_ref, o_ref):
    d = d_ref[...]
    xv = jnp.concatenate([xs_ref[qq] for qq in range(NQ)], axis=1)
    pv = jnp.concatenate([p_ref[qq] for qq in range(NQ)], axis=1)
    t = d * (xv + pv)
    h = jnp.dot(t, w1_ref[...], preferred_element_type=jnp.float32)
    h = jax.nn.gelu(h + b1_ref[...])
    y = jnp.dot(h, w2_ref[...], preferred_element_type=jnp.float32)
    yd = y * d
    for qq in range(NQ):
        o_ref[qq] = yd[:, DQ * qq:DQ * (qq + 1)]


def _mid_call(xs4, p4, dcol, W1, b1, W2):
    rb = 1000
    dh = W1.shape[1]
    return pl.pallas_call(
        _mid_body,
        grid=(N // rb,),
        in_specs=[
            pl.BlockSpec((NQ, rb, DQ), lambda i: (0, i, 0)),
            pl.BlockSpec((NQ, rb, DQ), lambda i: (0, i, 0)),
            pl.BlockSpec((rb, 1), lambda i: (i, 0)),
            pl.BlockSpec((D, dh), lambda i: (0, 0)),
            pl.BlockSpec((1, dh), lambda i: (0, 0)),
            pl.BlockSpec((dh, D), lambda i: (0, 0)),
        ],
        out_specs=pl.BlockSpec((NQ, rb, DQ), lambda i: (0, i, 0)),
        out_shape=jax.ShapeDtypeStruct((NQ, N, DQ), jnp.float32),
    )(xs4, p4, dcol, W1, b1.reshape(1, dh), W2)


def _fin_body(ys_ref, q_ref, d_ref, b2_ref, o_ref):
    yv = jnp.concatenate([ys_ref[qq] for qq in range(NQ)], axis=1)
    qv = jnp.concatenate([q_ref[qq] for qq in range(NQ)], axis=1)
    t = d_ref[...] * (yv + qv)
    o_ref[...] = jax.nn.gelu(t + b2_ref[...])


def _fin_call(ys4, q4, dcol, b2):
    rb = 1000
    return pl.pallas_call(
        _fin_body,
        grid=(N // rb,),
        in_specs=[
            pl.BlockSpec((NQ, rb, DQ), lambda i: (0, i, 0)),
            pl.BlockSpec((NQ, rb, DQ), lambda i: (0, i, 0)),
            pl.BlockSpec((rb, 1), lambda i: (i, 0)),
            pl.BlockSpec((1, D), lambda i: (0, 0)),
        ],
        out_specs=pl.BlockSpec((rb, D), lambda i: (i, 0)),
        out_shape=jax.ShapeDtypeStruct((N, D), jnp.float32),
    )(ys4, q4, dcol, b2.reshape(1, D))


def kernel(x, edge_index, batch, W1, b1, W2, b2):
    src = edge_index[0]
    dst = edge_index[1]
    pad = EPAD - E
    srcp = jnp.concatenate(
        [src, jnp.zeros((pad,), jnp.int32)]).reshape(NS, NCH, CB)
    dstp = jnp.concatenate(
        [dst, jnp.full((pad,), JUNK, jnp.int32)]).reshape(NS, NCH, CB)
    zrows = jnp.zeros((NPAD, DQ), jnp.float32)
    iot = jnp.arange(N, dtype=jnp.int32).reshape(NS, TRT // TGB, TGB)
    srcp, dstp, zrows, iot = lax.optimization_barrier((srcp, dstp, zrows, iot))

    deg_parts = _deg_call(dst)
    dinv = _dinv_call(deg_parts)
    dcol = dinv.reshape(NPAD, 1)[:N]

    xs4 = _scale_call(x, dcol)
    p4 = _agg_call(xs4, srcp, dstp, zrows, iot)
    ys4 = _mid_call(xs4, p4[:, :N], dcol, W1, b1, W2)
    q4 = _agg_call(ys4, srcp, dstp, zrows, iot)
    out = _fin_call(ys4, q4[:, :N], dcol, b2)
    return (out, None)
